# baseline (speedup 1.0000x reference)
import functools

import jax
import jax.numpy as jnp
from jax import lax
from jax.experimental import pallas as pl
from jax.experimental.pallas import tpu as pltpu
from jax.experimental.pallas import tpu_sc as plsc

_DIM = 32
_EPS = 1e-06
_MARGIN = 1.0


def _sc_body(bpw, nc, ent_hbm, rel_hbm, ph, pr, pt, nh, nr, nt,
             pos_out, neg_out,
             phv, prv, ptv, nhv, nrv, ntv, hp, rp, tp, hn, rn, tn, sem):
    wid = lax.axis_index("s") * nc + lax.axis_index("c")
    base = wid * bpw
    sl = pl.ds(base, bpw)
    pltpu.sync_copy(ph.at[sl], phv)
    pltpu.sync_copy(pr.at[sl], prv)
    pltpu.sync_copy(pt.at[sl], ptv)
    pltpu.sync_copy(nh.at[sl], nhv)
    pltpu.sync_copy(nr.at[sl], nrv)
    pltpu.sync_copy(nt.at[sl], ntv)
    copies = [
        pltpu.async_copy(ent_hbm.at[phv], hp, sem),
        pltpu.async_copy(rel_hbm.at[prv], rp, sem),
        pltpu.async_copy(ent_hbm.at[ptv], tp, sem),
        pltpu.async_copy(ent_hbm.at[nhv], hn, sem),
        pltpu.async_copy(rel_hbm.at[nrv], rn, sem),
        pltpu.async_copy(ent_hbm.at[ntv], tn, sem),
    ]
    for c in copies:
        c.wait()

    def row(i, carry):
        for col in (0, 16):
            csl = pl.ds(col, 16)
            hp[i, csl] = hp[i, csl] + rp[i, csl] - tp[i, csl] + _EPS
            hn[i, csl] = hn[i, csl] + rn[i, csl] - tn[i, csl] + _EPS
        return carry

    lax.fori_loop(0, bpw, row, 0, unroll=4)
    pltpu.sync_copy(hp, pos_out.at[sl, :])
    pltpu.sync_copy(hn, neg_out.at[sl, :])


@functools.lru_cache(maxsize=None)
def _make_sc_gather_diff(batch):
    info = plsc.get_sparse_core_info()
    nc, ns = info.num_cores, info.num_subcores
    nw = nc * ns
    assert batch % nw == 0
    bpw = batch // nw
    mesh = plsc.VectorSubcoreMesh(core_axis_name="c", subcore_axis_name="s")
    return pl.kernel(
        functools.partial(_sc_body, bpw, nc),
        out_type=[
            jax.ShapeDtypeStruct((batch, _DIM), jnp.float32),
            jax.ShapeDtypeStruct((batch, _DIM), jnp.float32),
        ],
        mesh=mesh,
        compiler_params=pltpu.CompilerParams(use_tc_tiling_on_sc=False),
        scratch_types=(
            [pltpu.VMEM((bpw,), jnp.int32)] * 6
            + [pltpu.VMEM((bpw, _DIM), jnp.float32)] * 6
            + [pltpu.SemaphoreType.DMA]
        ),
    )


def _tc_body(pd_ref, nd_ref, out_ref):
    pd = pd_ref[...]
    nd = nd_ref[...]
    ps = jnp.sum(pd * pd, axis=1)
    ns = jnp.sum(nd * nd, axis=1)
    hinge = jnp.maximum(jnp.sqrt(ps) - jnp.sqrt(ns) + _MARGIN, 0.0)
    out_ref[0, 0] = jnp.sum(hinge) / pd.shape[0]


def kernel(pos_x, neg_x, ent_emb, rel_emb):
    batch = pos_x.shape[0]
    ph, pr, pt = pos_x[:, 0], pos_x[:, 1], pos_x[:, 2]
    nh, nr, nt = neg_x[:, 0], neg_x[:, 1], neg_x[:, 2]
    pos_diff, neg_diff = _make_sc_gather_diff(batch)(
        ent_emb, rel_emb, ph, pr, pt, nh, nr, nt)
    out = pl.pallas_call(
        _tc_body,
        out_shape=jax.ShapeDtypeStruct((1, 1), jnp.float32),
        out_specs=pl.BlockSpec(memory_space=pltpu.SMEM),
    )(pos_diff, neg_diff)
    return out[0, 0]
